# baseline (speedup 1.0000x reference)
import functools

import jax
import jax.numpy as jnp
from jax.experimental import pallas as pl
from jax.experimental.pallas import tpu as pltpu

_B = 16384
_C = 100
_BS = 1024


def _loss_body(yhat_ref, yf_ref, f2c_ref, f2c_col_ref, out_ref, acc_ref,
               *, scale):
    x = yhat_ref[...]
    yf = yf_ref[...]
    f2c = f2c_ref[...]
    f2c_col = f2c_col_ref[...]

    bs, c = x.shape
    col = jax.lax.broadcasted_iota(jnp.int32, (bs, c), 1)
    onehot = (col == yf).astype(jnp.bfloat16)
    y_coarse = jax.lax.dot_general(
        onehot, f2c_col, (((1,), (0,)), ((), ())),
        preferred_element_type=jnp.float32)
    mask = f2c == y_coarse

    m = jnp.max(x, axis=1, keepdims=True)
    t = x - m
    e = jnp.exp(t)
    e_hi = e.astype(jnp.bfloat16)
    e_lo = (e - e_hi.astype(jnp.float32)).astype(jnp.bfloat16)
    zero_bf = jnp.zeros((), jnp.bfloat16)
    em_hi = jnp.where(mask, e_hi, zero_bf)
    em_lo = jnp.where(mask, e_lo, zero_bf)
    ones_col = jnp.ones((c, 1), jnp.bfloat16)
    dot = functools.partial(
        jax.lax.dot_general,
        dimension_numbers=(((1,), (0,)), ((), ())),
        preferred_element_type=jnp.float32)
    z = dot(e_hi, ones_col) + dot(e_lo, ones_col)
    e_in = dot(em_hi, ones_col) + dot(em_lo, ones_col)

    w = z * jnp.float32(1.0 + 1e-7) - e_in
    px = e * (1.0 / w)
    px_log = jnp.log(jnp.maximum(px, 1e-10))
    contrib = jnp.where(mask, 0.0, px * px_log)

    part = jnp.sum(contrib.reshape(bs // 8, 8, c), axis=0) * scale

    @pl.when(pl.program_id(0) == 0)
    def _():
        acc_ref[...] = jnp.zeros_like(acc_ref)

    acc_ref[...] += part

    @pl.when(pl.program_id(0) == pl.num_programs(0) - 1)
    def _():
        out_ref[0, 0] = jnp.sum(acc_ref[...])


@jax.jit
def kernel(yHat, y_fine, fine2coarse):
    b, c = yHat.shape
    scale = 1.0 / (float(b) * float(c))
    grid = b // _BS
    out = pl.pallas_call(
        functools.partial(_loss_body, scale=scale),
        grid=(grid,),
        in_specs=[
            pl.BlockSpec((_BS, c), lambda i: (i, 0)),
            pl.BlockSpec((_BS, 1), lambda i: (i, 0)),
            pl.BlockSpec((1, c), lambda i: (0, 0)),
            pl.BlockSpec((c, 1), lambda i: (0, 0)),
        ],
        out_specs=pl.BlockSpec(memory_space=pltpu.SMEM),
        out_shape=jax.ShapeDtypeStruct((1, 1), jnp.float32),
        scratch_shapes=[pltpu.VMEM((8, c), jnp.float32)],
    )(yHat, y_fine.reshape(b, 1),
      fine2coarse.astype(jnp.float32).reshape(1, c),
      fine2coarse.astype(jnp.bfloat16).reshape(c, 1))
    return out[0, 0]


# speedup vs baseline: 1.5088x; 1.5088x over previous
import functools

import jax
import jax.numpy as jnp
from jax.experimental import pallas as pl
from jax.experimental.pallas import tpu as pltpu

_B = 16384
_C = 100
_BS = 1024

_LOG_CLIP = -23.025850929940457


def _loss_body(yhat_ref, yf_ref, f2cb_ref, out_ref, acc_ref, *, scale):
    x = yhat_ref[...]
    yf = yf_ref[...]
    f2cb = f2cb_ref[...]

    xt = x.T
    c, bs = xt.shape
    rowi = jax.lax.broadcasted_iota(jnp.int32, (c, bs), 0)
    onehot = rowi == yf
    y_c = jnp.sum(jnp.where(onehot, f2cb, 0.0), axis=0, keepdims=True)
    mask = f2cb == y_c

    m = jnp.max(xt, axis=0, keepdims=True)
    t = xt - m
    e = jnp.exp(t)
    z = jnp.sum(e, axis=0, keepdims=True)
    e_in = jnp.sum(jnp.where(mask, e, 0.0), axis=0, keepdims=True)
    w = z * jnp.float32(1.0 + 1e-7) - e_in
    lw = jnp.log(w)
    term = jnp.maximum(t - lw, jnp.float32(_LOG_CLIP))
    cten = jnp.where(mask, 0.0, e * term)
    s = jnp.sum(cten, axis=0, keepdims=True) / w

    @pl.when(pl.program_id(0) == 0)
    def _():
        acc_ref[...] = jnp.zeros_like(acc_ref)

    acc_ref[...] += s * scale

    @pl.when(pl.program_id(0) == pl.num_programs(0) - 1)
    def _():
        out_ref[0, 0] = jnp.sum(acc_ref[...])


@jax.jit
def kernel(yHat, y_fine, fine2coarse):
    b, c = yHat.shape
    scale = 1.0 / (float(b) * float(c))
    grid = b // _BS
    f2cb = jnp.broadcast_to(
        fine2coarse.astype(jnp.float32).reshape(c, 1), (c, _BS))
    out = pl.pallas_call(
        functools.partial(_loss_body, scale=scale),
        grid=(grid,),
        in_specs=[
            pl.BlockSpec((_BS, c), lambda i: (i, 0)),
            pl.BlockSpec((1, _BS), lambda i: (0, i)),
            pl.BlockSpec((c, _BS), lambda i: (0, 0)),
        ],
        out_specs=pl.BlockSpec(memory_space=pltpu.SMEM),
        out_shape=jax.ShapeDtypeStruct((1, 1), jnp.float32),
        scratch_shapes=[pltpu.VMEM((1, _BS), jnp.float32)],
    )(yHat, y_fine.reshape(1, b), f2cb)
    return out[0, 0]
